# Initial kernel scaffold; baseline (speedup 1.0000x reference)
import jax
import jax.numpy as jnp
from jax.experimental import pallas as pl


def kernel(opcode_ids, operand_ids, opcode_table, operand_table):
    raise NotImplementedError("write your pallas kernel here")


# speedup vs baseline: 1.5519x; 1.5519x over previous
import jax
import jax.numpy as jnp
from jax import lax
from jax.experimental import pallas as pl
from jax.experimental.pallas import tpu as pltpu
from jax.experimental.pallas import tpu_sc as plsc

B, S, M, D = 1024, 200, 4, 64
N = B * S
NC, NS = 2, 16
NW = NC * NS
PER_W = N // NW
C = 128
NCH = PER_W // C


def _body(opd_ids_hbm, opc_ids_hbm, opd_tab_hbm, opc_tab_hbm, out_hbm,
          idv, icv, rows, oprows, outb, sem):
    wid = lax.axis_index("s") * NC + lax.axis_index("c")
    base = wid * PER_W

    def chunk(g, carry):
        cb = base + g * C
        for m in range(4):
            pltpu.sync_copy(opd_ids_hbm.at[pl.ds(m * N + cb, C)],
                            idv.at[pl.ds(m * C, C)])
        pltpu.sync_copy(opc_ids_hbm.at[pl.ds(cb, C)], icv)
        cps = [pltpu.async_copy(opd_tab_hbm.at[idv.at[pl.ds(m * C, C)]],
                                rows.at[pl.ds(m * C, C)], sem)
               for m in range(4)]
        cps.append(pltpu.async_copy(opc_tab_hbm.at[icv], oprows, sem))
        for cp in cps:
            cp.wait()

        def group(g2, c2):
            i0 = 16 * g2
            mvecs = [jnp.minimum(idv[pl.ds(m * C + i0, 16)], 1)
                     .astype(jnp.float32) for m in range(4)]
            cntv = mvecs[0] + mvecs[1] + mvecs[2] + mvecs[3]
            invv = 1.0 / (cntv + 1e-10)
            wvecs = [mk * invv for mk in mvecs]
            for k in range(16):
                i = i0 + k
                bidx = jnp.full((16,), k, jnp.int32)
                ws = [wvecs[m].at[bidx].get(mode="promise_in_bounds")
                      for m in range(4)]
                for j in range(4):
                    sl = pl.ds(16 * j, 16)
                    acc = oprows[i, sl]
                    for m in range(4):
                        acc = acc + rows[m * C + i, sl] * ws[m]
                    outb[i, sl] = acc
            return c2

        lax.fori_loop(0, C // 16, group, 0)
        pltpu.sync_copy(outb, out_hbm.at[pl.ds(cb, C)])
        return carry

    lax.fori_loop(0, NCH, chunk, 0)


def kernel(opcode_ids, operand_ids, opcode_table, operand_table):
    opc = opcode_ids.reshape(N).astype(jnp.int32)
    opd = operand_ids.reshape(N, M).T.reshape(M * N).astype(jnp.int32)
    mesh = plsc.VectorSubcoreMesh(core_axis_name="c", subcore_axis_name="s",
                                  num_cores=NC, num_subcores=NS)
    f = pl.kernel(
        _body,
        out_type=jax.ShapeDtypeStruct((N, D), jnp.float32),
        mesh=mesh,
        compiler_params=pltpu.CompilerParams(use_tc_tiling_on_sc=False),
        scratch_types=[
            pltpu.VMEM((4 * C,), jnp.int32),
            pltpu.VMEM((C,), jnp.int32),
            pltpu.VMEM((4 * C, D), jnp.float32),
            pltpu.VMEM((C, D), jnp.float32),
            pltpu.VMEM((C, D), jnp.float32),
            pltpu.SemaphoreType.DMA,
        ],
    )
    out = f(opd, opc, operand_table, opcode_table)
    return out.reshape(B, S, D)
